# baseline (speedup 1.0000x reference)
import jax
import jax.numpy as jnp
from jax.experimental import pallas as pl

_RATE = 0.1
_SCALE = float(jnp.float32(1.0) / jnp.float32(1.0 - _RATE))
_THRESH = 838861
_K0 = 0
_K1 = 42
_KS2 = (_K0 ^ _K1 ^ 0x1BD11BDA)

_ROT1 = (13, 15, 26, 6)
_ROT2 = (17, 29, 16, 24)

_BLK_R = 8
_BLK_C = 8192
_BLK = _BLK_R * _BLK_C


def _rotl(x, d):
    return jax.lax.shift_left(x, jnp.int32(d)) | jax.lax.shift_right_logical(
        x, jnp.int32(32 - d)
    )


def _mix(x0, x1, rots):
    for r in rots:
        x0 = x0 + x1
        x1 = _rotl(x1, r) ^ x0
    return x0, x1


def _threefry_bits(p):
    ks0 = jnp.int32(_K0)
    ks1 = jnp.int32(_K1)
    ks2 = jnp.int32(_KS2)
    x0 = jnp.zeros_like(p) + ks0
    x1 = p + ks1
    x0, x1 = _mix(x0, x1, _ROT1)
    x0, x1 = x0 + ks1, x1 + (ks2 + jnp.int32(1))
    x0, x1 = _mix(x0, x1, _ROT2)
    x0, x1 = x0 + ks2, x1 + (ks0 + jnp.int32(2))
    x0, x1 = _mix(x0, x1, _ROT1)
    x0, x1 = x0 + ks0, x1 + (ks1 + jnp.int32(3))
    x0, x1 = _mix(x0, x1, _ROT2)
    x0, x1 = x0 + ks1, x1 + (ks2 + jnp.int32(4))
    x0, x1 = _mix(x0, x1, _ROT1)
    x0, x1 = x0 + ks2, x1 + (ks0 + jnp.int32(5))
    return x0 ^ x1


def _body(v_ref, o_ref):
    b = pl.program_id(0)
    row = jax.lax.broadcasted_iota(jnp.int32, (_BLK_R, _BLK_C), 0)
    col = jax.lax.broadcasted_iota(jnp.int32, (_BLK_R, _BLK_C), 1)
    p = b * _BLK + row * _BLK_C + col
    bits = _threefry_bits(p)
    keep = jax.lax.shift_right_logical(bits, jnp.int32(9)) >= jnp.int32(_THRESH)
    v = v_ref[...]
    o_ref[...] = jnp.where(keep, v * _SCALE, jnp.float32(0.0))


def kernel(values, indices):
    nnz = values.shape[0]
    nblk = (nnz + _BLK - 1) // _BLK
    total = nblk * _BLK
    vp = jnp.pad(values, (0, total - nnz)).reshape(nblk * _BLK_R, _BLK_C)
    out = pl.pallas_call(
        _body,
        grid=(nblk,),
        in_specs=[pl.BlockSpec((_BLK_R, _BLK_C), lambda b: (b, 0))],
        out_specs=pl.BlockSpec((_BLK_R, _BLK_C), lambda b: (b, 0)),
        out_shape=jax.ShapeDtypeStruct((nblk * _BLK_R, _BLK_C), jnp.float32),
    )(vp)
    return out.reshape(-1)[:nnz], indices


# speedup vs baseline: 1.9241x; 1.9241x over previous
import jax
import jax.numpy as jnp
import numpy as np
from jax.experimental import pallas as pl

_RATE = 0.1
_SCALE = float(np.float32(1.0) / np.float32(1.0 - _RATE))
_THRESH = 838861
_K0 = 0
_K1 = 42
_KS2 = (_K0 ^ _K1 ^ 0x1BD11BDA)

_ROT1 = (13, 15, 26, 6)
_ROT2 = (17, 29, 16, 24)

_BLK_R2 = 512
_BLK_C2 = 128
_BLK = _BLK_R2 * _BLK_C2


def _rotl(x, d):
    return jax.lax.shift_left(x, jnp.int32(d)) | jax.lax.shift_right_logical(
        x, jnp.int32(32 - d)
    )


def _mix(x0, x1, rots):
    for r in rots:
        x0 = x0 + x1
        x1 = _rotl(x1, r) ^ x0
    return x0, x1


def _threefry_bits(p):
    ks0 = jnp.int32(_K0)
    ks1 = jnp.int32(_K1)
    ks2 = jnp.int32(_KS2)
    x0 = jnp.zeros_like(p) + ks0
    x1 = p + ks1
    x0, x1 = _mix(x0, x1, _ROT1)
    x0, x1 = x0 + ks1, x1 + (ks2 + jnp.int32(1))
    x0, x1 = _mix(x0, x1, _ROT2)
    x0, x1 = x0 + ks2, x1 + (ks0 + jnp.int32(2))
    x0, x1 = _mix(x0, x1, _ROT1)
    x0, x1 = x0 + ks0, x1 + (ks1 + jnp.int32(3))
    x0, x1 = _mix(x0, x1, _ROT2)
    x0, x1 = x0 + ks1, x1 + (ks2 + jnp.int32(4))
    x0, x1 = _mix(x0, x1, _ROT1)
    x0, x1 = x0 + ks2, x1 + (ks0 + jnp.int32(5))
    return x0 ^ x1


def _body(v_ref, o_ref):
    b = pl.program_id(0)
    row = jax.lax.broadcasted_iota(jnp.int32, (_BLK_R2, _BLK_C2), 0)
    col = jax.lax.broadcasted_iota(jnp.int32, (_BLK_R2, _BLK_C2), 1)
    p = b * _BLK + row * _BLK_C2 + col
    bits = _threefry_bits(p)
    keep = jax.lax.shift_right_logical(bits, jnp.int32(9)) >= jnp.int32(_THRESH)
    v = v_ref[...].reshape(_BLK_R2, _BLK_C2)
    res = jnp.where(keep, v * _SCALE, jnp.float32(0.0))
    o_ref[...] = res.reshape(_BLK)


def kernel(values, indices):
    nnz = values.shape[0]
    nblk = (nnz + _BLK - 1) // _BLK
    out = pl.pallas_call(
        _body,
        grid=(nblk,),
        in_specs=[pl.BlockSpec((_BLK,), lambda b: (b,))],
        out_specs=pl.BlockSpec((_BLK,), lambda b: (b,)),
        out_shape=jax.ShapeDtypeStruct((nnz,), jnp.float32),
    )(values)
    return out, indices
